# baseline (speedup 1.0000x reference)
import functools

import jax
import jax.numpy as jnp
from jax import lax
from jax.experimental import pallas as pl
from jax.experimental.pallas import tpu as pltpu
from jax.experimental.pallas import tpu_sc as plsc

N = 10000
E = 320000
D = 128
G = 64

NC = 2
NS = 16
NW = NC * NS
EC = 80
RT = E // NW // EC
NQ = EC // 16
SEG = 5
SR = RT // SEG
NPT = 640
NPT_LAST = N - NPT * (NS - 1)
RBLK = 2000
NBLK = N // RBLK


def _wid(c, s):
    return c * NS + s


def _copy_tile_slice(s, src, dst):
    @pl.when(s < NS - 1)
    def _():
        pltpu.sync_copy(src.at[pl.ds(s * NPT, NPT)],
                        dst.at[pl.ds(s * NPT, NPT)])

    @pl.when(s == NS - 1)
    def _():
        pltpu.sync_copy(src.at[pl.ds((NS - 1) * NPT, NPT_LAST)],
                        dst.at[pl.ds((NS - 1) * NPT, NPT_LAST)])


def _stage_idx_row(idx_v, j, idx_cur):
    for q in range(NQ):
        idx_cur[pl.ds(q * 16, 16)] = idx_v[j, pl.ds(q * 16, 16)]


def _deg_body(dst3d, zeros128, ones128, out, idx_v, idx_cur, idx_cur1,
              ones_v, sem0, sem1, acc):
    c = lax.axis_index("c")
    s = lax.axis_index("s")
    w = _wid(c, s)
    pltpu.sync_copy(dst3d.at[w], idx_v)
    pltpu.sync_copy(ones128, ones_v)
    _copy_tile_slice(s, zeros128, acc)
    plsc.subcore_barrier()

    _stage_idx_row(idx_v, 0, idx_cur)
    pltpu.async_copy(ones_v, acc.at[idx_cur], sem0, add=True)

    def step(j2, carry):
        j = j2 * 2
        _stage_idx_row(idx_v, j + 1, idx_cur1)
        pltpu.async_copy(ones_v, acc.at[idx_cur1], sem1, add=True)
        pltpu.make_async_copy(ones_v, acc.at[idx_cur], sem0).wait()
        _stage_idx_row(idx_v, j + 2, idx_cur)
        pltpu.async_copy(ones_v, acc.at[idx_cur], sem0, add=True)
        pltpu.make_async_copy(ones_v, acc.at[idx_cur1], sem1).wait()
        return carry

    lax.fori_loop(0, (RT - 1) // 2, step, 0)
    pltpu.make_async_copy(ones_v, acc.at[idx_cur], sem0).wait()
    plsc.subcore_barrier()
    _copy_tile_slice(s, acc, out.at[c])


@functools.cache
def _deg_kernel():
    return pl.kernel(
        _deg_body,
        out_type=jax.ShapeDtypeStruct((NC, N, D), jnp.float32),
        mesh=plsc.VectorSubcoreMesh(core_axis_name="c", subcore_axis_name="s",
                                    num_cores=NC, num_subcores=NS),
        scratch_types=[
            pltpu.VMEM((RT, EC), jnp.int32),
            pltpu.VMEM((EC,), jnp.int32),
            pltpu.VMEM((EC,), jnp.int32),
            pltpu.VMEM((EC, D), jnp.float32),
            pltpu.SemaphoreType.DMA,
            pltpu.SemaphoreType.DMA,
            pltpu.VMEM_SHARED((N, D), jnp.float32),
        ],
    )


def _edge_body(hn, src4d, dst4d, zeros128, out, src_v, dst_v,
               src_cur0, src_cur1, src_cur2, dst_cur0, dst_cur1, dst_cur2,
               rows0, rows1, rows2, sem0, sem1, sem2,
               ssem0, ssem1, ssem2, acc):
    c = lax.axis_index("c")
    s = lax.axis_index("s")
    w = _wid(c, s)
    _copy_tile_slice(s, zeros128, acc)
    plsc.subcore_barrier()

    def gath(sv, j, cur, rows, gsem):
        _stage_idx_row(sv, j, cur)
        pltpu.async_copy(hn.at[cur], rows, gsem)

    def scat(dv, j, scur, gcur, rows, gsem, ssem):
        pltpu.make_async_copy(hn.at[gcur], rows, gsem).wait()
        _stage_idx_row(dv, j, scur)
        pltpu.async_copy(rows, acc.at[scur], ssem, add=True)

    def swait(scur, rows, ssem):
        pltpu.make_async_copy(rows, acc.at[scur], ssem).wait()

    def segment(seg, carry):
        pltpu.sync_copy(src4d.at[w, seg], src_v)
        pltpu.sync_copy(dst4d.at[w, seg], dst_v)
        gath(src_v, 0, src_cur0, rows0, sem0)
        gath(src_v, 1, src_cur1, rows1, sem1)

        def step(t, carry2):
            j = t * 3

            @pl.when(t > 0)
            def _():
                swait(dst_cur2, rows2, ssem2)

            gath(src_v, j + 2, src_cur2, rows2, sem2)
            scat(dst_v, j, dst_cur0, src_cur0, rows0, sem0, ssem0)
            scat(dst_v, j + 1, dst_cur1, src_cur1, rows1, sem1, ssem1)
            swait(dst_cur0, rows0, ssem0)
            gath(src_v, j + 3, src_cur0, rows0, sem0)
            scat(dst_v, j + 2, dst_cur2, src_cur2, rows2, sem2, ssem2)
            swait(dst_cur1, rows1, ssem1)

            @pl.when(t < (SR - 1) // 3 - 1)
            def _():
                gath(src_v, j + 4, src_cur1, rows1, sem1)

            return carry2

        lax.fori_loop(0, (SR - 1) // 3, step, 0)
        scat(dst_v, SR - 1, dst_cur0, src_cur0, rows0, sem0, ssem0)
        swait(dst_cur2, rows2, ssem2)
        swait(dst_cur0, rows0, ssem0)
        return carry

    lax.fori_loop(0, SEG, segment, 0)
    plsc.subcore_barrier()
    _copy_tile_slice(s, acc, out.at[c])


@functools.cache
def _edge_kernel():
    return pl.kernel(
        _edge_body,
        out_type=jax.ShapeDtypeStruct((NC, N, D), jnp.float32),
        mesh=plsc.VectorSubcoreMesh(core_axis_name="c", subcore_axis_name="s",
                                    num_cores=NC, num_subcores=NS),
        scratch_types=[
            pltpu.VMEM((SR, EC), jnp.int32),
            pltpu.VMEM((SR, EC), jnp.int32),
            pltpu.VMEM((EC,), jnp.int32),
            pltpu.VMEM((EC,), jnp.int32),
            pltpu.VMEM((EC,), jnp.int32),
            pltpu.VMEM((EC,), jnp.int32),
            pltpu.VMEM((EC,), jnp.int32),
            pltpu.VMEM((EC,), jnp.int32),
            pltpu.VMEM((EC, D), jnp.float32),
            pltpu.VMEM((EC, D), jnp.float32),
            pltpu.VMEM((EC, D), jnp.float32),
            pltpu.SemaphoreType.DMA,
            pltpu.SemaphoreType.DMA,
            pltpu.SemaphoreType.DMA,
            pltpu.SemaphoreType.DMA,
            pltpu.SemaphoreType.DMA,
            pltpu.SemaphoreType.DMA,
            pltpu.VMEM_SHARED((N, D), jnp.float32),
        ],
    )


def _dinv_from_deg(deg_blk):
    deg = jnp.sum(deg_blk[0] + deg_blk[1], axis=-1,
                  keepdims=True) * (1.0 / D) + 1.0
    return lax.rsqrt(deg)


def _mm1_body(x_ref, w_ref, deg_ref, hn_ref, dinv_ref):
    dinv = _dinv_from_deg(deg_ref[...])
    dinv_ref[...] = dinv
    hn_ref[...] = jnp.dot(x_ref[...], w_ref[...],
                          preferred_element_type=jnp.float32) * dinv


def _mm1(x, w0, deg_part):
    return pl.pallas_call(
        _mm1_body,
        grid=(NBLK,),
        in_specs=[
            pl.BlockSpec((RBLK, D), lambda i: (i, 0)),
            pl.BlockSpec((D, D), lambda i: (0, 0)),
            pl.BlockSpec((NC, RBLK, D), lambda i: (0, i, 0)),
        ],
        out_specs=[pl.BlockSpec((RBLK, D), lambda i: (i, 0)),
                   pl.BlockSpec((RBLK, 1), lambda i: (i, 0))],
        out_shape=[jax.ShapeDtypeStruct((N, D), jnp.float32),
                   jax.ShapeDtypeStruct((N, 1), jnp.float32)],
    )(x, w0, deg_part)


def _mm2_body(part_ref, hn_ref, dinv_ref, b_ref, w_ref, out_ref):
    dinv = dinv_ref[...]
    t = part_ref[0] + part_ref[1] + hn_ref[...]
    t = jnp.maximum(t * dinv + b_ref[...], 0.0)
    out_ref[...] = jnp.dot(t, w_ref[...],
                           preferred_element_type=jnp.float32) * dinv


def _mm2(part, hn, dinv, b2d, w1):
    return pl.pallas_call(
        _mm2_body,
        grid=(NBLK,),
        in_specs=[
            pl.BlockSpec((NC, RBLK, D), lambda i: (0, i, 0)),
            pl.BlockSpec((RBLK, D), lambda i: (i, 0)),
            pl.BlockSpec((RBLK, 1), lambda i: (i, 0)),
            pl.BlockSpec((1, D), lambda i: (0, 0)),
            pl.BlockSpec((D, D), lambda i: (0, 0)),
        ],
        out_specs=pl.BlockSpec((RBLK, D), lambda i: (i, 0)),
        out_shape=jax.ShapeDtypeStruct((N, D), jnp.float32),
    )(part, hn, dinv, b2d, w1)


def _head_body(bounds_ref, part_ref, hn_ref, dinv_ref, b_ref, batch_ref,
               wf_ref, bf_ref, out_ref, gmax_ref):
    i = pl.program_id(0)

    @pl.when(i == 0)
    def _():
        gmax_ref[...] = jnp.full((G, D), -jnp.inf, jnp.float32)

    dinv = dinv_ref[...]
    h = part_ref[0] + part_ref[1] + hn_ref[...]
    h = jnp.maximum(h * dinv + b_ref[...], 0.0)
    batch = batch_ref[0]
    neg = jnp.float32(-jnp.inf)
    lo = bounds_ref[i, 0]
    hi = bounds_ref[i, 1]
    for g in range(G):
        @pl.when((lo <= g) & (g <= hi))
        def _():
            m = jnp.max(jnp.where(batch == g, h, neg), axis=0, keepdims=True)
            gmax_ref[g:g + 1, :] = jnp.maximum(gmax_ref[g:g + 1, :], m)

    @pl.when(i == NBLK - 1)
    def _():
        out_ref[...] = jnp.maximum(
            jnp.dot(gmax_ref[...], wf_ref[...],
                    preferred_element_type=jnp.float32) + bf_ref[...],
            0.0)


def _head(bounds, part, hn, dinv, b2d, batch3d, wf, bf2d):
    return pl.pallas_call(
        _head_body,
        grid_spec=pltpu.PrefetchScalarGridSpec(
            num_scalar_prefetch=1,
            grid=(NBLK,),
            in_specs=[
                pl.BlockSpec((NC, RBLK, D), lambda i, b: (0, i, 0)),
                pl.BlockSpec((RBLK, D), lambda i, b: (i, 0)),
                pl.BlockSpec((RBLK, 1), lambda i, b: (i, 0)),
                pl.BlockSpec((1, D), lambda i, b: (0, 0)),
                pl.BlockSpec((1, RBLK, 1), lambda i, b: (i, 0, 0)),
                pl.BlockSpec((D, D), lambda i, b: (0, 0)),
                pl.BlockSpec((1, D), lambda i, b: (0, 0)),
            ],
            out_specs=pl.BlockSpec((G, D), lambda i, b: (0, 0)),
            scratch_shapes=[pltpu.VMEM((G, D), jnp.float32)],
        ),
        out_shape=jax.ShapeDtypeStruct((G, D), jnp.float32),
    )(bounds, part, hn, dinv, b2d, batch3d, wf, bf2d)


def kernel(x, edge_index, batch, W0, b0, W1, b1, Wf, bf):
    src4d = edge_index[0].reshape(NW, SEG, SR, EC)
    dst4d = edge_index[1].reshape(NW, SEG, SR, EC)
    dst3d = edge_index[1].reshape(NW, RT, EC)
    ones128 = jnp.ones((EC, D), jnp.float32)
    zeros128 = jnp.zeros((N, D), jnp.float32)
    batch3d = batch.reshape(NBLK, RBLK, 1)

    deg_part = _deg_kernel()(dst3d, zeros128, ones128)
    hn1, dinv = _mm1(x, W0, deg_part)
    part1 = _edge_kernel()(hn1, src4d, dst4d, zeros128)
    hn2 = _mm2(part1, hn1, dinv, b0.reshape(1, D), W1)
    part2 = _edge_kernel()(hn2, src4d, dst4d, zeros128)
    bounds = jnp.stack([batch[::RBLK], batch[RBLK - 1::RBLK]], axis=1)
    out = _head(bounds, part2, hn2, dinv, b1.reshape(1, D), batch3d,
                Wf, bf.reshape(1, D))
    return out


# speedup vs baseline: 1.1385x; 1.1385x over previous
import functools

import jax
import jax.numpy as jnp
from jax import lax
from jax.experimental import pallas as pl
from jax.experimental.pallas import tpu as pltpu
from jax.experimental.pallas import tpu_sc as plsc

N = 10000
E = 320000
D = 128
G = 64

NC = 2
NS = 16
NW = NC * NS
EC = 80
RT = E // NW // EC
NQ = EC // 16
SEG = 5
SR = RT // SEG
NPT = 640
NPT_LAST = N - NPT * (NS - 1)
RBLK = 2000
NBLK = N // RBLK


def _wid(c, s):
    return c * NS + s


def _copy_tile_slice(s, src, dst):
    @pl.when(s < NS - 1)
    def _():
        pltpu.sync_copy(src.at[pl.ds(s * NPT, NPT)],
                        dst.at[pl.ds(s * NPT, NPT)])

    @pl.when(s == NS - 1)
    def _():
        pltpu.sync_copy(src.at[pl.ds((NS - 1) * NPT, NPT_LAST)],
                        dst.at[pl.ds((NS - 1) * NPT, NPT_LAST)])


def _stage_idx_row(idx_v, j, idx_cur):
    for q in range(NQ):
        idx_cur[pl.ds(q * 16, 16)] = idx_v[j, pl.ds(q * 16, 16)]


def _deg_body(dst3d, zeros128, ones128, out, idx_v, idx_cur, idx_cur1,
              ones_v, sem0, sem1, acc):
    c = lax.axis_index("c")
    s = lax.axis_index("s")
    w = _wid(c, s)
    pltpu.sync_copy(dst3d.at[w], idx_v)
    pltpu.sync_copy(ones128, ones_v)
    _copy_tile_slice(s, zeros128, acc)
    plsc.subcore_barrier()

    _stage_idx_row(idx_v, 0, idx_cur)
    pltpu.async_copy(ones_v, acc.at[idx_cur], sem0, add=True)

    def step(j2, carry):
        j = j2 * 2
        _stage_idx_row(idx_v, j + 1, idx_cur1)
        pltpu.async_copy(ones_v, acc.at[idx_cur1], sem1, add=True)
        pltpu.make_async_copy(ones_v, acc.at[idx_cur], sem0).wait()
        _stage_idx_row(idx_v, j + 2, idx_cur)
        pltpu.async_copy(ones_v, acc.at[idx_cur], sem0, add=True)
        pltpu.make_async_copy(ones_v, acc.at[idx_cur1], sem1).wait()
        return carry

    lax.fori_loop(0, (RT - 1) // 2, step, 0)
    pltpu.make_async_copy(ones_v, acc.at[idx_cur], sem0).wait()
    plsc.subcore_barrier()
    _copy_tile_slice(s, acc, out.at[c])


@functools.cache
def _deg_kernel():
    return pl.kernel(
        _deg_body,
        out_type=jax.ShapeDtypeStruct((NC, N, D), jnp.float32),
        mesh=plsc.VectorSubcoreMesh(core_axis_name="c", subcore_axis_name="s",
                                    num_cores=NC, num_subcores=NS),
        scratch_types=[
            pltpu.VMEM((RT, EC), jnp.int32),
            pltpu.VMEM((EC,), jnp.int32),
            pltpu.VMEM((EC,), jnp.int32),
            pltpu.VMEM((EC, D), jnp.float32),
            pltpu.SemaphoreType.DMA,
            pltpu.SemaphoreType.DMA,
            pltpu.VMEM_SHARED((N, D), jnp.float32),
        ],
    )


def _edge_body(hn, src4d, dst4d, zeros128, out, src_v, dst_v,
               src_cur0, src_cur1, src_cur2, dst_cur0, dst_cur1, dst_cur2,
               rows0, rows1, rows2, sem0, sem1, sem2,
               ssem0, ssem1, ssem2, acc):
    c = lax.axis_index("c")
    s = lax.axis_index("s")
    w = _wid(c, s)
    _copy_tile_slice(s, zeros128, acc)
    plsc.subcore_barrier()

    def gath(sv, j, cur, rows, gsem):
        _stage_idx_row(sv, j, cur)
        pltpu.async_copy(hn.at[cur], rows, gsem)

    def scat(dv, j, scur, gcur, rows, gsem, ssem):
        pltpu.make_async_copy(hn.at[gcur], rows, gsem).wait()
        _stage_idx_row(dv, j, scur)
        pltpu.async_copy(rows, acc.at[scur], ssem, add=True)

    def swait(scur, rows, ssem):
        pltpu.make_async_copy(rows, acc.at[scur], ssem).wait()

    def segment(seg, carry):
        pltpu.sync_copy(src4d.at[w, seg], src_v)
        pltpu.sync_copy(dst4d.at[w, seg], dst_v)
        gath(src_v, 0, src_cur0, rows0, sem0)
        gath(src_v, 1, src_cur1, rows1, sem1)

        def step(t, carry2):
            j = t * 3

            @pl.when(t > 0)
            def _():
                swait(dst_cur2, rows2, ssem2)

            gath(src_v, j + 2, src_cur2, rows2, sem2)
            scat(dst_v, j, dst_cur0, src_cur0, rows0, sem0, ssem0)
            swait(dst_cur0, rows0, ssem0)
            gath(src_v, j + 3, src_cur0, rows0, sem0)
            scat(dst_v, j + 1, dst_cur1, src_cur1, rows1, sem1, ssem1)
            swait(dst_cur1, rows1, ssem1)

            @pl.when(t < (SR - 1) // 3 - 1)
            def _():
                gath(src_v, j + 4, src_cur1, rows1, sem1)

            scat(dst_v, j + 2, dst_cur2, src_cur2, rows2, sem2, ssem2)
            return carry2

        lax.fori_loop(0, (SR - 1) // 3, step, 0)
        scat(dst_v, SR - 1, dst_cur0, src_cur0, rows0, sem0, ssem0)
        swait(dst_cur2, rows2, ssem2)
        swait(dst_cur0, rows0, ssem0)
        return carry

    lax.fori_loop(0, SEG, segment, 0)
    plsc.subcore_barrier()
    _copy_tile_slice(s, acc, out.at[c])


@functools.cache
def _edge_kernel():
    return pl.kernel(
        _edge_body,
        out_type=jax.ShapeDtypeStruct((NC, N, D), jnp.float32),
        mesh=plsc.VectorSubcoreMesh(core_axis_name="c", subcore_axis_name="s",
                                    num_cores=NC, num_subcores=NS),
        scratch_types=[
            pltpu.VMEM((SR, EC), jnp.int32),
            pltpu.VMEM((SR, EC), jnp.int32),
            pltpu.VMEM((EC,), jnp.int32),
            pltpu.VMEM((EC,), jnp.int32),
            pltpu.VMEM((EC,), jnp.int32),
            pltpu.VMEM((EC,), jnp.int32),
            pltpu.VMEM((EC,), jnp.int32),
            pltpu.VMEM((EC,), jnp.int32),
            pltpu.VMEM((EC, D), jnp.float32),
            pltpu.VMEM((EC, D), jnp.float32),
            pltpu.VMEM((EC, D), jnp.float32),
            pltpu.SemaphoreType.DMA,
            pltpu.SemaphoreType.DMA,
            pltpu.SemaphoreType.DMA,
            pltpu.SemaphoreType.DMA,
            pltpu.SemaphoreType.DMA,
            pltpu.SemaphoreType.DMA,
            pltpu.VMEM_SHARED((N, D), jnp.float32),
        ],
    )


def _dinv_from_deg(deg_blk):
    deg = jnp.sum(deg_blk[0] + deg_blk[1], axis=-1,
                  keepdims=True) * (1.0 / D) + 1.0
    return lax.rsqrt(deg)


def _mm1_body(x_ref, w_ref, deg_ref, hn_ref, dinv_ref):
    dinv = _dinv_from_deg(deg_ref[...])
    dinv_ref[...] = dinv
    hn_ref[...] = jnp.dot(x_ref[...], w_ref[...],
                          preferred_element_type=jnp.float32) * dinv


def _mm1(x, w0, deg_part):
    return pl.pallas_call(
        _mm1_body,
        grid=(NBLK,),
        in_specs=[
            pl.BlockSpec((RBLK, D), lambda i: (i, 0)),
            pl.BlockSpec((D, D), lambda i: (0, 0)),
            pl.BlockSpec((NC, RBLK, D), lambda i: (0, i, 0)),
        ],
        out_specs=[pl.BlockSpec((RBLK, D), lambda i: (i, 0)),
                   pl.BlockSpec((RBLK, 1), lambda i: (i, 0))],
        out_shape=[jax.ShapeDtypeStruct((N, D), jnp.float32),
                   jax.ShapeDtypeStruct((N, 1), jnp.float32)],
    )(x, w0, deg_part)


def _mm2_body(part_ref, hn_ref, dinv_ref, b_ref, w_ref, out_ref):
    dinv = dinv_ref[...]
    t = part_ref[0] + part_ref[1] + hn_ref[...]
    t = jnp.maximum(t * dinv + b_ref[...], 0.0)
    out_ref[...] = jnp.dot(t, w_ref[...],
                           preferred_element_type=jnp.float32) * dinv


def _mm2(part, hn, dinv, b2d, w1):
    return pl.pallas_call(
        _mm2_body,
        grid=(NBLK,),
        in_specs=[
            pl.BlockSpec((NC, RBLK, D), lambda i: (0, i, 0)),
            pl.BlockSpec((RBLK, D), lambda i: (i, 0)),
            pl.BlockSpec((RBLK, 1), lambda i: (i, 0)),
            pl.BlockSpec((1, D), lambda i: (0, 0)),
            pl.BlockSpec((D, D), lambda i: (0, 0)),
        ],
        out_specs=pl.BlockSpec((RBLK, D), lambda i: (i, 0)),
        out_shape=jax.ShapeDtypeStruct((N, D), jnp.float32),
    )(part, hn, dinv, b2d, w1)


def _head_body(bounds_ref, part_ref, hn_ref, dinv_ref, b_ref, batch_ref,
               wf_ref, bf_ref, out_ref, gmax_ref):
    i = pl.program_id(0)

    @pl.when(i == 0)
    def _():
        gmax_ref[...] = jnp.full((G, D), -jnp.inf, jnp.float32)

    dinv = dinv_ref[...]
    h = part_ref[0] + part_ref[1] + hn_ref[...]
    h = jnp.maximum(h * dinv + b_ref[...], 0.0)
    batch = batch_ref[0]
    neg = jnp.float32(-jnp.inf)
    lo = bounds_ref[i, 0]
    hi = bounds_ref[i, 1]
    for g in range(G):
        @pl.when((lo <= g) & (g <= hi))
        def _():
            m = jnp.max(jnp.where(batch == g, h, neg), axis=0, keepdims=True)
            gmax_ref[g:g + 1, :] = jnp.maximum(gmax_ref[g:g + 1, :], m)

    @pl.when(i == NBLK - 1)
    def _():
        out_ref[...] = jnp.maximum(
            jnp.dot(gmax_ref[...], wf_ref[...],
                    preferred_element_type=jnp.float32) + bf_ref[...],
            0.0)


def _head(bounds, part, hn, dinv, b2d, batch3d, wf, bf2d):
    return pl.pallas_call(
        _head_body,
        grid_spec=pltpu.PrefetchScalarGridSpec(
            num_scalar_prefetch=1,
            grid=(NBLK,),
            in_specs=[
                pl.BlockSpec((NC, RBLK, D), lambda i, b: (0, i, 0)),
                pl.BlockSpec((RBLK, D), lambda i, b: (i, 0)),
                pl.BlockSpec((RBLK, 1), lambda i, b: (i, 0)),
                pl.BlockSpec((1, D), lambda i, b: (0, 0)),
                pl.BlockSpec((1, RBLK, 1), lambda i, b: (i, 0, 0)),
                pl.BlockSpec((D, D), lambda i, b: (0, 0)),
                pl.BlockSpec((1, D), lambda i, b: (0, 0)),
            ],
            out_specs=pl.BlockSpec((G, D), lambda i, b: (0, 0)),
            scratch_shapes=[pltpu.VMEM((G, D), jnp.float32)],
        ),
        out_shape=jax.ShapeDtypeStruct((G, D), jnp.float32),
    )(bounds, part, hn, dinv, b2d, batch3d, wf, bf2d)


def kernel(x, edge_index, batch, W0, b0, W1, b1, Wf, bf):
    src4d = edge_index[0].reshape(NW, SEG, SR, EC)
    dst4d = edge_index[1].reshape(NW, SEG, SR, EC)
    dst3d = edge_index[1].reshape(NW, RT, EC)
    ones128 = jnp.ones((EC, D), jnp.float32)
    zeros128 = jnp.zeros((N, D), jnp.float32)
    batch3d = batch.reshape(NBLK, RBLK, 1)

    deg_part = _deg_kernel()(dst3d, zeros128, ones128)
    hn1, dinv = _mm1(x, W0, deg_part)
    part1 = _edge_kernel()(hn1, src4d, dst4d, zeros128)
    hn2 = _mm2(part1, hn1, dinv, b0.reshape(1, D), W1)
    part2 = _edge_kernel()(hn2, src4d, dst4d, zeros128)
    bounds = jnp.stack([batch[::RBLK], batch[RBLK - 1::RBLK]], axis=1)
    out = _head(bounds, part2, hn2, dinv, b1.reshape(1, D), batch3d,
                Wf, bf.reshape(1, D))
    return out
